# Initial kernel scaffold; baseline (speedup 1.0000x reference)
import jax
import jax.numpy as jnp
from jax.experimental import pallas as pl


def kernel(boxes, scores):
    raise NotImplementedError("write your pallas kernel here")


# speedup vs baseline: 34.7574x; 34.7574x over previous
import jax
import jax.numpy as jnp
from jax.experimental import pallas as pl
from jax.experimental.pallas import tpu as pltpu

_N = 20000
_MAX_DET = 200
_SCORE_THRESH = 0.5
_NMS_THRESH = 0.2
_L = 128
_R = 160
_NP = _R * _L
_NEG = -1e9


def _nms_body(x1_ref, y1_ref, x2_ref, y2_ref, sc_ref, out_ref, sw_ref):
    x1 = x1_ref[...]
    y1 = y1_ref[...]
    x2 = x2_ref[...]
    y2 = y2_ref[...]
    area = (x2 - x1) * (y2 - y1)
    s = sc_ref[...]
    sw = jnp.where(s > _SCORE_THRESH, s, _NEG)
    sw_ref[...] = sw

    rows = jax.lax.broadcasted_iota(jnp.int32, (_R, _L), 0)
    lanes = jax.lax.broadcasted_iota(jnp.int32, (_R, _L), 1)
    lin = rows * _L + lanes
    lane1 = jax.lax.broadcasted_iota(jnp.int32, (1, _L), 1)

    m0 = jnp.max(sw)
    idx0 = jnp.min(jnp.where(sw == m0, lin, jnp.int32(_NP)))

    def body(i, carry):
        idx, best = carry
        r = idx // _L
        c = idx % _L

        def ext(ref):
            row = ref[pl.ds(r, 1), :]
            return jnp.sum(jnp.where(lane1 == c, row, jnp.float32(0.0)))

        bx1 = ext(x1_ref)
        by1 = ext(y1_ref)
        bx2 = ext(x2_ref)
        by2 = ext(y2_ref)
        barea = (bx2 - bx1) * (by2 - by1)
        valid = best > 0.0

        xx1 = jnp.maximum(bx1, x1)
        yy1 = jnp.maximum(by1, y1)
        xx2 = jnp.minimum(bx2, x2)
        yy2 = jnp.minimum(by2, y2)
        inter = jnp.maximum(xx2 - xx1, 0.0) * jnp.maximum(yy2 - yy1, 0.0)
        iou = inter / (barea + area - inter + 1e-9)
        suppress = jnp.logical_or(iou > _NMS_THRESH, lin == idx)

        sw_cur = sw_ref[...]
        new_sw = jnp.where(jnp.logical_and(valid, suppress), _NEG, sw_cur)
        sw_ref[...] = new_sw

        m = jnp.max(new_sw)
        nidx = jnp.min(jnp.where(new_sw == m, lin, jnp.int32(_NP)))

        row = jnp.where(
            lane1 == 0, bx1,
            jnp.where(lane1 == 1, by1,
                      jnp.where(lane1 == 2, bx2,
                                jnp.where(lane1 == 3, by2,
                                          jnp.where(lane1 == 4, best,
                                                    jnp.float32(0.0))))))
        row = row * valid.astype(jnp.float32)
        out_ref[pl.ds(i, 1), :] = row
        return (nidx, m)

    jax.lax.fori_loop(0, _MAX_DET, body, (idx0, m0))


def kernel(boxes, scores):
    pad = _NP - _N
    x1 = jnp.pad(boxes[:, 0], (0, pad)).reshape(_R, _L)
    y1 = jnp.pad(boxes[:, 1], (0, pad)).reshape(_R, _L)
    x2 = jnp.pad(boxes[:, 2], (0, pad)).reshape(_R, _L)
    y2 = jnp.pad(boxes[:, 3], (0, pad)).reshape(_R, _L)
    s = jnp.pad(scores, (0, pad)).reshape(_R, _L)

    out = pl.pallas_call(
        _nms_body,
        out_shape=jax.ShapeDtypeStruct((_MAX_DET, _L), jnp.float32),
        scratch_shapes=[pltpu.VMEM((_R, _L), jnp.float32)],
    )(x1, y1, x2, y2, s)
    return out[:, :5]
